# baseline (speedup 1.0000x reference)
import functools

import jax
import jax.numpy as jnp
from jax import lax
from jax.experimental import pallas as pl
from jax.experimental.pallas import tpu as pltpu
from jax.experimental.pallas import tpu_sc as plsc

L = 16
NC, NS = 2, 16
NW = NC * NS
BATCH = 256
H, W = 513, 257
HM, WM = H - 1, W - 1
HS = 16
NSTRIPE = HM // HS
WV = WM // L
LP = W + HM
LPP = 784
LPV = LPP // L
NSH = 257
NSP = 272
RPW = BATCH // NW
EPS = 1e-5


def _main_body(x_hbm, w_hbm, idx_hbm, out_hbm,
               x_buf, w_buf, idx_buf,
               a0, a1, a2, a3, a4, a5, a6, a7,
               out_buf, sem):
    accs = (a0, a1, a2, a3, a4, a5, a6, a7)
    wid = lax.axis_index("s") * NC + lax.axis_index("c")
    row0 = wid * RPW

    zeros = jnp.zeros((L,), jnp.float32)

    def zbody(i, c):
        o = i * L
        for r in range(RPW):
            accs[r][pl.ds(o, L)] = zeros
        return c

    lax.fori_loop(0, NSP // L, zbody, 0)

    def chunk_dmas(s, slot):
        h0 = pl.multiple_of(s * HS, HS)
        return (
            pltpu.make_async_copy(
                x_hbm.at[pl.ds(row0, RPW), pl.ds(h0, HS), pl.ds(0, WM)],
                x_buf.at[slot], sem),
            pltpu.make_async_copy(
                w_hbm.at[pl.ds(h0, HS), pl.ds(0, WM)], w_buf.at[slot], sem),
            pltpu.make_async_copy(
                idx_hbm.at[pl.ds(h0, HS), pl.ds(0, WM)], idx_buf.at[slot],
                sem),
        )

    def start(s, slot):
        for d in chunk_dmas(s, slot):
            d.start()

    def wait(s, slot):
        for d in chunk_dmas(s, slot):
            d.wait()

    def compute(slot):
        @plsc.parallel_loop(0, HS)
        def hbody(hh):
            for v in range(WV):
                o = v * L
                wv = w_buf[slot, hh, pl.ds(o, L)]
                iv = idx_buf[slot, hh, pl.ds(o, L)]
                for r in range(RPW):
                    xv = x_buf[slot, r, hh, pl.ds(o, L)]
                    yv = xv * xv * wv
                    plsc.addupdate_scatter(accs[r], [iv], yv)

    start(0, 0)

    def cbody(s, carry):
        slot = lax.rem(s, 2)
        wait(s, slot)

        @pl.when(s + 1 < NSTRIPE)
        def _():
            start(s + 1, 1 - slot)

        compute(slot)
        return carry

    lax.fori_loop(0, NSTRIPE, cbody, 0)

    for r in range(RPW):
        for v in range(NSP // L):
            o = v * L
            out_buf[r, pl.ds(o, L)] = accs[r][pl.ds(o, L)]
    pltpu.sync_copy(out_buf, out_hbm.at[pl.ds(row0, RPW)])


def _tail_body(sums_hbm, xl_hbm, wl_hbm, il_hbm, cnt_hbm, out_hbm,
               xl_buf, wl_buf, il_buf,
               a0, a1, a2, a3, a4, a5, a6, a7,
               cnt_buf, rec, out_buf):
    accs = (a0, a1, a2, a3, a4, a5, a6, a7)
    wid = lax.axis_index("s") * NC + lax.axis_index("c")
    row0 = wid * RPW

    pltpu.sync_copy(sums_hbm.at[pl.ds(row0, RPW)], out_buf)
    pltpu.sync_copy(xl_hbm.at[pl.ds(row0, RPW)], xl_buf)
    pltpu.sync_copy(wl_hbm, wl_buf)
    pltpu.sync_copy(il_hbm, il_buf)
    pltpu.sync_copy(cnt_hbm, cnt_buf)

    zeros = jnp.zeros((L,), jnp.float32)
    for v in range(NSP // L):
        o = v * L
        for r in range(RPW):
            accs[r][pl.ds(o, L)] = zeros

    @plsc.parallel_loop(0, LPV)
    def lbody(i):
        o = i * L
        wv = wl_buf[pl.ds(o, L)]
        iv = il_buf[pl.ds(o, L)]
        for r in range(RPW):
            xv = xl_buf[r, pl.ds(o, L)]
            yv = xv * xv * wv
            plsc.addupdate_scatter(accs[r], [iv], yv)

    for v in range(NSP // L):
        o = v * L
        rec[pl.ds(o, L)] = 1.0 / (cnt_buf[pl.ds(o, L)] + EPS)
    for r in range(RPW):
        for v in range(NSP // L):
            o = v * L
            out_buf[r, pl.ds(o, L)] = (
                (out_buf[r, pl.ds(o, L)] + accs[r][pl.ds(o, L)])
                * rec[pl.ds(o, L)])
    pltpu.sync_copy(out_buf, out_hbm.at[pl.ds(row0, RPW)])


@jax.jit
def _sc_spectrum(x3, w2, idx2, xl, wl, il, cnt):
    mesh = plsc.VectorSubcoreMesh(core_axis_name="c", subcore_axis_name="s")
    params = pltpu.CompilerParams(needs_layout_passes=False)
    main = pl.kernel(
        _main_body,
        mesh=mesh,
        compiler_params=params,
        out_type=jax.ShapeDtypeStruct((BATCH, NSP), jnp.float32),
        scratch_types=(
            [
                pltpu.VMEM((2, RPW, HS, WM), jnp.float32),
                pltpu.VMEM((2, HS, WM), jnp.float32),
                pltpu.VMEM((2, HS, WM), jnp.int32),
            ]
            + [pltpu.VMEM((NSP,), jnp.float32) for _ in range(RPW)]
            + [
                pltpu.VMEM((RPW, NSP), jnp.float32),
                pltpu.SemaphoreType.DMA,
            ]
        ),
    )
    tail = pl.kernel(
        _tail_body,
        mesh=mesh,
        compiler_params=params,
        out_type=jax.ShapeDtypeStruct((BATCH, NSP), jnp.float32),
        scratch_types=(
            [
                pltpu.VMEM((RPW, LPP), jnp.float32),
                pltpu.VMEM((LPP,), jnp.float32),
                pltpu.VMEM((LPP,), jnp.int32),
            ]
            + [pltpu.VMEM((NSP,), jnp.float32) for _ in range(RPW)]
            + [
                pltpu.VMEM((NSP,), jnp.float32),
                pltpu.VMEM((NSP,), jnp.float32),
                pltpu.VMEM((RPW, NSP), jnp.float32),
            ]
        ),
    )
    sums = main(x3, w2, idx2)
    return tail(sums, xl, wl, il, cnt)


def kernel(x, shells_weight, shell_index, shells_count):
    b, c, h, w_ = x.shape
    x3 = lax.optimization_barrier(x.reshape(b, h, w_))
    xl = jnp.concatenate([x3[:, h - 1, :], x3[:, : h - 1, w_ - 1]], axis=1)
    xl = jnp.pad(xl, ((0, 0), (0, LPP - LP)))
    wl = jnp.concatenate(
        [shells_weight[h - 1, :], shells_weight[: h - 1, w_ - 1],
         jnp.zeros((LPP - LP,), jnp.float32)])
    il = jnp.concatenate(
        [shell_index[h - 1, :], shell_index[: h - 1, w_ - 1],
         jnp.zeros((LPP - LP,), jnp.int32)])
    cnt = jnp.concatenate(
        [shells_count, jnp.ones((NSP - NSH,), jnp.float32)])
    out = _sc_spectrum(x3, shells_weight, shell_index, xl, wl, il, cnt)
    return out[:, :NSH].reshape(b, c, NSH)


# speedup vs baseline: 1.5211x; 1.5211x over previous
import functools

import jax
import jax.numpy as jnp
from jax import lax
from jax.experimental import pallas as pl
from jax.experimental.pallas import tpu as pltpu
from jax.experimental.pallas import tpu_sc as plsc

L = 16
NC, NS = 2, 16
NW = NC * NS
BATCH = 256
H, W = 513, 257
HM, WM = H - 1, W - 1
HS = 16
NSTRIPE = HM // HS
WV = WM // L
LP = W + HM
LPP = 784
LPV = LPP // L
NSH = 257
NSP = 272
RPW = BATCH // NW
EPS = 1e-5


def _main_body(x_hbm, w_hbm, idx_hbm, out_hbm,
               x_buf, w_buf, idx_buf,
               a0, a1, a2, a3, a4, a5, a6, a7,
               out_buf, sem):
    accs = (a0, a1, a2, a3, a4, a5, a6, a7)
    wid = lax.axis_index("s") * NC + lax.axis_index("c")
    row0 = wid * RPW

    zeros = jnp.zeros((L,), jnp.float32)

    def zbody(i, c):
        o = i * L
        for r in range(RPW):
            accs[r][pl.ds(o, L)] = zeros
        return c

    lax.fori_loop(0, NSP // L, zbody, 0)

    def chunk_dmas(s, slot):
        h0 = pl.multiple_of(s * HS, HS)
        return (
            pltpu.make_async_copy(
                x_hbm.at[pl.ds(row0, RPW), pl.ds(h0, HS), pl.ds(0, WM)],
                x_buf.at[slot], sem),
            pltpu.make_async_copy(
                w_hbm.at[pl.ds(h0, HS), pl.ds(0, WM)], w_buf.at[slot], sem),
            pltpu.make_async_copy(
                idx_hbm.at[pl.ds(h0, HS), pl.ds(0, WM)], idx_buf.at[slot],
                sem),
        )

    def start(s, slot):
        for d in chunk_dmas(s, slot):
            d.start()

    def wait(s, slot):
        for d in chunk_dmas(s, slot):
            d.wait()

    def compute(slot):
        @plsc.parallel_loop(0, HS)
        def hbody(hh):
            for v in range(WV):
                o = v * L
                wv = w_buf[slot, hh, pl.ds(o, L)]
                iv = idx_buf[slot, hh, pl.ds(o, L)]
                for r in range(RPW):
                    xv = x_buf[slot, r, hh, pl.ds(o, L)]
                    yv = xv * xv * wv
                    plsc.addupdate_scatter(accs[r], [iv], yv)

    start(0, 0)

    def cbody(s, carry):
        slot = lax.rem(s, 2)
        wait(s, slot)

        @pl.when(s + 1 < NSTRIPE)
        def _():
            start(s + 1, 1 - slot)

        compute(slot)
        return carry

    lax.fori_loop(0, NSTRIPE, cbody, 0)

    for r in range(RPW):
        for v in range(NSP // L):
            o = v * L
            out_buf[r, pl.ds(o, L)] = accs[r][pl.ds(o, L)]
    pltpu.sync_copy(out_buf, out_hbm.at[pl.ds(row0, RPW)])


def _tail_body(sums_hbm, xl_hbm, wl_hbm, il_hbm, cnt_hbm, out_hbm,
               xl_buf, wl_buf, il_buf,
               a0, a1, a2, a3, a4, a5, a6, a7,
               cnt_buf, rec, out_buf):
    accs = (a0, a1, a2, a3, a4, a5, a6, a7)
    wid = lax.axis_index("s") * NC + lax.axis_index("c")
    row0 = wid * RPW

    pltpu.sync_copy(sums_hbm.at[pl.ds(row0, RPW)], out_buf)
    pltpu.sync_copy(xl_hbm.at[pl.ds(row0, RPW)], xl_buf)
    pltpu.sync_copy(wl_hbm, wl_buf)
    pltpu.sync_copy(il_hbm, il_buf)
    pltpu.sync_copy(cnt_hbm, cnt_buf)

    zeros = jnp.zeros((L,), jnp.float32)
    for v in range(NSP // L):
        o = v * L
        for r in range(RPW):
            accs[r][pl.ds(o, L)] = zeros

    @plsc.parallel_loop(0, LPV)
    def lbody(i):
        o = i * L
        wv = wl_buf[pl.ds(o, L)]
        iv = il_buf[pl.ds(o, L)]
        for r in range(RPW):
            xv = xl_buf[r, pl.ds(o, L)]
            yv = xv * xv * wv
            plsc.addupdate_scatter(accs[r], [iv], yv)

    for v in range(NSP // L):
        o = v * L
        rec[pl.ds(o, L)] = 1.0 / (cnt_buf[pl.ds(o, L)] + EPS)
    for r in range(RPW):
        for v in range(NSP // L):
            o = v * L
            out_buf[r, pl.ds(o, L)] = (
                (out_buf[r, pl.ds(o, L)] + accs[r][pl.ds(o, L)])
                * rec[pl.ds(o, L)])
    pltpu.sync_copy(out_buf, out_hbm.at[pl.ds(row0, RPW)])


@jax.jit
def _sc_spectrum(x3, w2, idx2, xl, wl, il, cnt):
    mesh = plsc.VectorSubcoreMesh(core_axis_name="c", subcore_axis_name="s")
    params = pltpu.CompilerParams(needs_layout_passes=False)
    main = pl.kernel(
        _main_body,
        mesh=mesh,
        compiler_params=params,
        out_type=jax.ShapeDtypeStruct((BATCH, NSP), jnp.float32),
        scratch_types=(
            [
                pltpu.VMEM((2, RPW, HS, WM), jnp.float32),
                pltpu.VMEM((2, HS, WM), jnp.float32),
                pltpu.VMEM((2, HS, WM), jnp.int32),
            ]
            + [pltpu.VMEM((NSP,), jnp.float32) for _ in range(RPW)]
            + [
                pltpu.VMEM((RPW, NSP), jnp.float32),
                pltpu.SemaphoreType.DMA,
            ]
        ),
    )
    tail = pl.kernel(
        _tail_body,
        mesh=mesh,
        compiler_params=params,
        out_type=jax.ShapeDtypeStruct((BATCH, NSP), jnp.float32),
        scratch_types=(
            [
                pltpu.VMEM((RPW, LPP), jnp.float32),
                pltpu.VMEM((LPP,), jnp.float32),
                pltpu.VMEM((LPP,), jnp.int32),
            ]
            + [pltpu.VMEM((NSP,), jnp.float32) for _ in range(RPW)]
            + [
                pltpu.VMEM((NSP,), jnp.float32),
                pltpu.VMEM((NSP,), jnp.float32),
                pltpu.VMEM((RPW, NSP), jnp.float32),
            ]
        ),
    )
    sums = main(x3, w2, idx2)
    return tail(sums, xl, wl, il, cnt)


def kernel(x, shells_weight, shell_index, shells_count):
    b, c, h, w_ = x.shape
    x3 = x.reshape(b, h, w_)
    xl = jnp.concatenate(
        [x[:, 0, h - 1, :], x[:, 0, : h - 1, w_ - 1]], axis=1)
    xl = jnp.pad(xl, ((0, 0), (0, LPP - LP)))
    wl = jnp.concatenate(
        [shells_weight[h - 1, :], shells_weight[: h - 1, w_ - 1],
         jnp.zeros((LPP - LP,), jnp.float32)])
    il = jnp.concatenate(
        [shell_index[h - 1, :], shell_index[: h - 1, w_ - 1],
         jnp.zeros((LPP - LP,), jnp.int32)])
    cnt = jnp.concatenate(
        [shells_count, jnp.ones((NSP - NSH,), jnp.float32)])
    out = _sc_spectrum(x3, shells_weight, shell_index, xl, wl, il, cnt)
    return out[:, :NSH].reshape(b, c, NSH)
